# baseline (speedup 1.0000x reference)
import functools

import jax
import jax.numpy as jnp
from jax import lax
from jax.experimental import pallas as pl
from jax.experimental.pallas import tpu as pltpu
from jax.experimental.pallas import tpu_sc as plsc

_VOCAB = 1000
_D = 1000
_DP = 1024
_B = 4096 * 20
_NC = 2
_NS = 16
_NW = _NC * _NS
_BPW = _B // _NW
_C = 16
_NCHUNK = _BPW // _C
_NBUF = 2

_mesh = plsc.VectorSubcoreMesh(core_axis_name="c", subcore_axis_name="s")


@functools.partial(
    pl.kernel,
    mesh=_mesh,
    out_type=jax.ShapeDtypeStruct((_B, _D), jnp.float32),
    scratch_types=[
        pltpu.VMEM((_BPW,), jnp.int32),
        pltpu.VMEM((_NBUF, _C, 8, 128), jnp.float32),
        pltpu.VMEM((_NBUF, _C, _D), jnp.float32),
        pltpu.SemaphoreType.DMA((_NBUF,)),
        pltpu.SemaphoreType.DMA((_NBUF,)),
    ],
)
def _gather_kernel(x_hbm, table_hbm, out_hbm, idx_v, rows_v, rb_v,
                   gsem, ssem):
    wid = lax.axis_index("s") * _NC + lax.axis_index("c")
    base = pl.multiple_of(wid * _BPW, 8)
    pltpu.sync_copy(x_hbm.at[pl.ds(base, _BPW)], idx_v)

    def gather_start(g, slot):
        off = pl.multiple_of(g * _C, 8)
        pltpu.async_copy(
            table_hbm.at[idx_v.at[pl.ds(off, _C)]],
            rows_v.at[slot],
            gsem.at[slot],
        )

    def gather_wait(slot):
        pltpu.make_async_copy(
            table_hbm.at[idx_v.at[pl.ds(0, _C)]],
            rows_v.at[slot],
            gsem.at[slot],
        ).wait()

    def out_start(g, slot):
        pltpu.async_copy(
            rb_v.at[slot],
            out_hbm.at[pl.ds(base + g * _C, _C)],
            ssem.at[slot],
        )

    def out_wait(slot):
        pltpu.make_async_copy(
            rb_v.at[slot],
            out_hbm.at[pl.ds(base, _C)],
            ssem.at[slot],
        ).wait()

    lane = lax.iota(jnp.int32, 16)

    def rearrange(slot):
        def per_row(p, _):
            for j in range(7):
                for v in range(0, 128, 16):
                    seg = rows_v[slot, p, j, pl.ds(v, 16)]
                    rb_v[slot, p, pl.ds(128 * j + v, 16)] = seg
            for v in range(0, 96, 16):
                seg = rows_v[slot, p, 7, pl.ds(v, 16)]
                rb_v[slot, p, pl.ds(896 + v, 16)] = seg
            seg = rows_v[slot, p, 7, pl.ds(96, 16)]
            rb_v[slot, p, pl.ds(992, 8)] = lax.slice(seg, (0,), (8,))
            return 0

        lax.fori_loop(0, _C, per_row, 0)

    for b in range(_NBUF):
        gather_start(b, b)

    def body(g, _):
        slot = lax.rem(g, _NBUF)
        gather_wait(slot)

        @pl.when(g >= _NBUF)
        def _():
            out_wait(slot)

        rearrange(slot)
        out_start(g, slot)

        @pl.when(g + _NBUF < _NCHUNK)
        def _():
            gather_start(g + _NBUF, slot)

        return 0

    lax.fori_loop(0, _NCHUNK, body, 0)

    for b in range(_NBUF):
        out_wait(b)


def _retile_tc(tbl_ref, out_ref):
    for j in range(7):
        out_ref[:, j, :] = tbl_ref[:, 128 * j:128 * (j + 1)]
    out_ref[:, 7, :] = jnp.pad(tbl_ref[:, 896:1000], ((0, 0), (0, 24)))


_retile = pl.pallas_call(
    _retile_tc,
    grid=(_VOCAB // 8,),
    in_specs=[pl.BlockSpec((8, _D), lambda i: (i, 0))],
    out_specs=pl.BlockSpec((8, 8, 128), lambda i: (i, 0, 0)),
    out_shape=jax.ShapeDtypeStruct((_VOCAB, 8, 128), jnp.float32),
)


def kernel(x, table):
    x_flat = x.reshape(-1).astype(jnp.int32)
    table3 = _retile(table)
    out = _gather_kernel(x_flat, table3)
    return out.reshape(x.shape[0], x.shape[1], _D)


# speedup vs baseline: 1.0472x; 1.0472x over previous
import functools

import jax
import jax.numpy as jnp
from jax import lax
from jax.experimental import pallas as pl
from jax.experimental.pallas import tpu as pltpu
from jax.experimental.pallas import tpu_sc as plsc

_VOCAB = 1000
_D = 1000
_DP = 1024
_B = 4096 * 20
_NC = 2
_NS = 16
_NW = _NC * _NS
_BPW = _B // _NW
_C = 16
_NCHUNK = _BPW // _C
_NBUF = 2

_mesh = plsc.VectorSubcoreMesh(core_axis_name="c", subcore_axis_name="s")


@functools.partial(
    pl.kernel,
    mesh=_mesh,
    out_type=jax.ShapeDtypeStruct((_B, _D), jnp.float32),
    scratch_types=[
        pltpu.VMEM((_BPW,), jnp.int32),
        pltpu.VMEM((_NBUF, _C, 8, 128), jnp.float32),
        pltpu.VMEM((_NBUF, _C, _D), jnp.float32),
        pltpu.SemaphoreType.DMA((_NBUF,)),
        pltpu.SemaphoreType.DMA((_NBUF,)),
    ],
)
def _gather_kernel(x_hbm, table_hbm, out_hbm, idx_v, rows_v, rb_v,
                   gsem, ssem):
    wid = lax.axis_index("s") * _NC + lax.axis_index("c")
    base = pl.multiple_of(wid * _BPW, 8)
    pltpu.sync_copy(x_hbm.at[pl.ds(base, _BPW)], idx_v)

    def gather_start(g, slot):
        off = pl.multiple_of(g * _C, 8)
        pltpu.async_copy(
            table_hbm.at[idx_v.at[pl.ds(off, _C)]],
            rows_v.at[slot],
            gsem.at[slot],
        )

    def gather_wait(slot):
        pltpu.make_async_copy(
            table_hbm.at[idx_v.at[pl.ds(0, _C)]],
            rows_v.at[slot],
            gsem.at[slot],
        ).wait()

    def out_start(g, slot):
        pltpu.async_copy(
            rb_v.at[slot],
            out_hbm.at[pl.ds(base + g * _C, _C)],
            ssem.at[slot],
        )

    def out_wait(slot):
        pltpu.make_async_copy(
            rb_v.at[slot],
            out_hbm.at[pl.ds(base, _C)],
            ssem.at[slot],
        ).wait()

    lane = lax.iota(jnp.int32, 16)

    def rearrange(slot):
        def per_row(p, _):
            for j in range(7):
                for v in range(0, 128, 16):
                    seg = rows_v[slot, p, j, pl.ds(v, 16)]
                    rb_v[slot, p, pl.ds(128 * j + v, 16)] = seg
            for v in range(0, 96, 16):
                seg = rows_v[slot, p, 7, pl.ds(v, 16)]
                rb_v[slot, p, pl.ds(896 + v, 16)] = seg
            seg = rows_v[slot, p, 7, pl.ds(96, 16)]
            rb_v[slot, p, pl.ds(992, 8)] = lax.slice(seg, (0,), (8,))
            return 0

        lax.fori_loop(0, _C, per_row, 0)

    for b in range(_NBUF):
        gather_start(b, b)

    def body(g, _):
        slot = lax.rem(g, _NBUF)
        gather_wait(slot)

        @pl.when(g >= _NBUF)
        def _():
            out_wait(slot)

        rearrange(slot)
        out_start(g, slot)

        @pl.when(g + _NBUF < _NCHUNK)
        def _():
            gather_start(g + _NBUF, slot)

        return 0

    lax.fori_loop(0, _NCHUNK, body, 0)

    for b in range(_NBUF):
        out_wait(b)


def _retile_tc(tbl_ref, out_ref):
    for j in range(7):
        out_ref[:, j, :] = tbl_ref[:, 128 * j:128 * (j + 1)]
    out_ref[:, 7, :] = jnp.pad(tbl_ref[:, 896:1000], ((0, 0), (0, 24)))


_retile = pl.pallas_call(
    _retile_tc,
    out_shape=jax.ShapeDtypeStruct((_VOCAB, 8, 128), jnp.float32),
)


def kernel(x, table):
    x_flat = x.reshape(-1).astype(jnp.int32)
    table3 = _retile(table)
    out = _gather_kernel(x_flat, table3)
    return out.reshape(x.shape[0], x.shape[1], _D)


# speedup vs baseline: 1.3316x; 1.2716x over previous
import functools

import jax
import jax.numpy as jnp
from jax import lax
from jax.experimental import pallas as pl
from jax.experimental.pallas import tpu as pltpu
from jax.experimental.pallas import tpu_sc as plsc

_VOCAB = 1000
_D = 1000
_B4 = 4096
_H = 20
_B = _B4 * _H
_NC = 2
_NS = 16
_NW = _NC * _NS
_PPW = _B4 // _NW
_RPW = _PPW * _H
_C = 16
_NCHUNK = _RPW // _C
_NSUP = _PPW // 4
_NBUF = 2
_NPB = 3

_GROUPS = (
    ((0, 16, 0, 0),),
    ((0, 4, 0, 16), (4, 12, 1, 0)),
    ((0, 8, 1, 12), (8, 8, 2, 0)),
    ((0, 12, 2, 8), (12, 4, 3, 0)),
    ((0, 16, 3, 4),),
)

_mesh = plsc.VectorSubcoreMesh(core_axis_name="c", subcore_axis_name="s")


@functools.partial(
    pl.kernel,
    mesh=_mesh,
    out_type=jax.ShapeDtypeStruct((_B4, _H, _D), jnp.float32),
    scratch_types=[
        pltpu.VMEM((_RPW,), jnp.int32),
        pltpu.VMEM((_NBUF, _C, 8, 128), jnp.float32),
        pltpu.VMEM((_NPB, _H, _D), jnp.float32),
        pltpu.SemaphoreType.DMA((_NBUF,)),
        pltpu.SemaphoreType.DMA((_NPB,)),
    ],
)
def _gather_kernel(x_hbm, table_hbm, out_hbm, idx_v, rows_v, rb_v,
                   gsem, ssem):
    wid = lax.axis_index("s") * _NC + lax.axis_index("c")
    pbase = wid * _PPW
    ibase = pl.multiple_of(pbase * _H, 8)
    pltpu.sync_copy(x_hbm.at[pl.ds(ibase, _RPW)], idx_v)

    def gather_start(g, slot):
        off = pl.multiple_of(g * _C, 16)
        pltpu.async_copy(
            table_hbm.at[idx_v.at[pl.ds(off, _C)]],
            rows_v.at[slot],
            gsem.at[slot],
        )

    def gather_wait(slot):
        pltpu.make_async_copy(
            table_hbm.at[idx_v.at[pl.ds(0, _C)]],
            rows_v.at[slot],
            gsem.at[slot],
        ).wait()

    def out_start(plane, buf):
        pltpu.async_copy(
            rb_v.at[buf],
            out_hbm.at[pbase + plane],
            ssem.at[buf],
        )

    def out_wait(buf):
        pltpu.make_async_copy(
            rb_v.at[buf],
            out_hbm.at[pbase],
            ssem.at[buf],
        ).wait()

    def rearrange(slot, src_lo, n, buf, q0):
        def per_row(i, _):
            p = src_lo + i
            q = q0 + i
            for j in range(7):
                for v in range(0, 128, 16):
                    seg = rows_v[slot, p, j, pl.ds(v, 16)]
                    rb_v[buf, q, pl.ds(128 * j + v, 16)] = seg
            for v in range(0, 96, 16):
                seg = rows_v[slot, p, 7, pl.ds(v, 16)]
                rb_v[buf, q, pl.ds(896 + v, 16)] = seg
            seg = rows_v[slot, p, 7, pl.ds(96, 16)]
            rb_v[buf, q, pl.ds(992, 8)] = lax.slice(seg, (0,), (8,))
            return 0

        lax.fori_loop(0, n, per_row, 0)

    for b in range(_NBUF):
        gather_start(b, b)

    def body(s, _):
        for c in range(5):
            g = s * 5 + c
            slot = lax.rem(g, _NBUF)
            gather_wait(slot)
            for (src_lo, n, j, q0) in _GROUPS[c]:
                plane = s * 4 + j
                buf = lax.rem(plane, _NPB)
                if q0 == 0:
                    @pl.when(plane >= _NPB)
                    def _(buf=buf):
                        out_wait(buf)
                rearrange(slot, src_lo, n, buf, q0)
                if q0 + n == _H:
                    out_start(plane, buf)

            @pl.when(g + _NBUF < _NCHUNK)
            def _(g=g, slot=slot):
                gather_start(g + _NBUF, slot)

        return 0

    lax.fori_loop(0, _NSUP, body, 0)

    for b in range(_NPB):
        out_wait(b)


def _retile_tc(tbl_ref, out_ref):
    for j in range(7):
        out_ref[:, j, :] = tbl_ref[:, 128 * j:128 * (j + 1)]
    out_ref[:, 7, :] = jnp.pad(tbl_ref[:, 896:1000], ((0, 0), (0, 24)))


_retile = pl.pallas_call(
    _retile_tc,
    out_shape=jax.ShapeDtypeStruct((_VOCAB, 8, 128), jnp.float32),
)


def kernel(x, table):
    x_flat = x.reshape(-1).astype(jnp.int32)
    table3 = _retile(table)
    return _gather_kernel(x_flat, table3)


# speedup vs baseline: 1.6863x; 1.2664x over previous
import functools

import jax
import jax.numpy as jnp
from jax import lax
from jax.experimental import pallas as pl
from jax.experimental.pallas import tpu as pltpu
from jax.experimental.pallas import tpu_sc as plsc

_VOCAB = 1000
_D = 1000
_B4 = 4096
_H = 20
_B = _B4 * _H
_NC = 2
_NS = 16
_NW = _NC * _NS
_PPW = _B4 // _NW
_RPW = _PPW * _H
_C = 16
_NCHUNK = _RPW // _C
_NSUP = _PPW // 4
_NBUF = 2
_NPB = 3

_GROUPS = (
    ((0, 16, 0, 0),),
    ((0, 4, 0, 16), (4, 12, 1, 0)),
    ((0, 8, 1, 12), (8, 8, 2, 0)),
    ((0, 12, 2, 8), (12, 4, 3, 0)),
    ((0, 16, 3, 4),),
)

_mesh = plsc.VectorSubcoreMesh(core_axis_name="c", subcore_axis_name="s")


@functools.partial(
    pl.kernel,
    mesh=_mesh,
    out_type=jax.ShapeDtypeStruct((_B4, _H, _D), jnp.float32),
    scratch_types=[
        pltpu.VMEM((_RPW,), jnp.int32),
        pltpu.VMEM((_NBUF, _C, 8, 128), jnp.float32),
        pltpu.VMEM((_NPB, _H, _D), jnp.float32),
        pltpu.SemaphoreType.DMA((_NBUF,)),
        pltpu.SemaphoreType.DMA((_NPB,)),
    ],
)
def _gather_kernel(x_hbm, table_hbm, out_hbm, idx_v, rows_v, rb_v,
                   gsem, ssem):
    wid = lax.axis_index("s") * _NC + lax.axis_index("c")
    pbase = wid * _PPW
    ibase = pl.multiple_of(pbase * _H, 8)
    pltpu.sync_copy(x_hbm.at[pl.ds(ibase, _RPW)], idx_v)

    def gather_start(g, slot):
        off = pl.multiple_of(g * _C, 16)
        pltpu.async_copy(
            table_hbm.at[idx_v.at[pl.ds(off, _C)]],
            rows_v.at[slot],
            gsem.at[slot],
        )

    def gather_wait(slot):
        pltpu.make_async_copy(
            table_hbm.at[idx_v.at[pl.ds(0, _C)]],
            rows_v.at[slot],
            gsem.at[slot],
        ).wait()

    def out_start(plane, buf):
        pltpu.async_copy(
            rb_v.at[buf],
            out_hbm.at[pbase + plane],
            ssem.at[buf],
        )

    def out_wait(buf):
        pltpu.make_async_copy(
            rb_v.at[buf],
            out_hbm.at[pbase],
            ssem.at[buf],
        ).wait()

    segmap = [(j, v, 128 * j + v) for j in range(7) for v in range(0, 128, 16)]
    segmap += [(7, v, 896 + v) for v in range(0, 96, 16)]

    def rearrange(slot, src_lo, n, buf, q0):
        @plsc.parallel_loop(0, n, unroll=4)
        def per_row(i):
            p = src_lo + i
            q = q0 + i
            for lo in range(0, len(segmap), 8):
                batch = segmap[lo:lo + 8]
                segs = [rows_v[slot, p, j, pl.ds(v, 16)] for j, v, _ in batch]
                for (j, v, off), seg in zip(batch, segs):
                    rb_v[buf, q, pl.ds(off, 16)] = seg
            seg = rows_v[slot, p, 7, pl.ds(96, 16)]
            rb_v[buf, q, pl.ds(992, 8)] = lax.slice(seg, (0,), (8,))

    for b in range(_NBUF):
        gather_start(b, b)

    def body(s, _):
        for c in range(5):
            g = s * 5 + c
            slot = lax.rem(g, _NBUF)
            gather_wait(slot)
            for (src_lo, n, j, q0) in _GROUPS[c]:
                plane = s * 4 + j
                buf = lax.rem(plane, _NPB)
                if q0 == 0:
                    @pl.when(plane >= _NPB)
                    def _(buf=buf):
                        out_wait(buf)
                rearrange(slot, src_lo, n, buf, q0)
                if q0 + n == _H:
                    out_start(plane, buf)

            @pl.when(g + _NBUF < _NCHUNK)
            def _(g=g, slot=slot):
                gather_start(g + _NBUF, slot)

        return 0

    lax.fori_loop(0, _NSUP, body, 0)

    for b in range(_NPB):
        out_wait(b)


def _retile_tc(tbl_ref, out_ref):
    for j in range(7):
        out_ref[:, j, :] = tbl_ref[:, 128 * j:128 * (j + 1)]
    out_ref[:, 7, :] = jnp.pad(tbl_ref[:, 896:1000], ((0, 0), (0, 24)))


_retile = pl.pallas_call(
    _retile_tc,
    out_shape=jax.ShapeDtypeStruct((_VOCAB, 8, 128), jnp.float32),
)


def kernel(x, table):
    x_flat = x.reshape(-1).astype(jnp.int32)
    table3 = _retile(table)
    return _gather_kernel(x_flat, table3)
